# Initial kernel scaffold; baseline (speedup 1.0000x reference)
import jax
import jax.numpy as jnp
from jax.experimental import pallas as pl


def kernel(node_features, edge_index, W_in, b_in, W_conv, b_conv, gamma, beta, W_pose, b_pose, W_cls, b_cls):
    raise NotImplementedError("write your pallas kernel here")


# speedup vs baseline: 5.2442x; 5.2442x over previous
import functools

import jax
import jax.numpy as jnp
from jax import lax
from jax.experimental import pallas as pl
from jax.experimental.pallas import tpu as pltpu
from jax.experimental.pallas import tpu_sc as plsc

N = 10000
E = 160000
DIN = 256
H = 512
NL = 12
OUT = 3
NC = 60

NCH = 4
CW = H // NCH
BW = 125
EROWS = E // BW
NTILE = 16
ROWS_PT = EROWS // NTILE
HROWS = ROWS_PT // 2
SPAD = 10240
STRIPE = SPAD // NTILE
CHK = 80
NZ = STRIPE // CHK
BN = 1000
NB = N // BN

@functools.cache
def _mesh():
    return plsc.VectorSubcoreMesh(core_axis_name="c", subcore_axis_name="s")


def _sc_deg_body(src2d, dst2d, ones_b, zdeg, deg, hist, idxv, onesv, zv, dumpv):
    core = lax.axis_index("c")
    sid = lax.axis_index("s")
    pltpu.sync_copy(ones_b, onesv)
    pltpu.sync_copy(zdeg, zv)
    for k in range(NZ):
        pltpu.sync_copy(zv, hist.at[pl.ds(sid * STRIPE + k * CHK, CHK)])

    def run(idx2d, outidx):
        pltpu.sync_copy(idx2d.at[pl.ds(sid * ROWS_PT, ROWS_PT)], idxv)
        plsc.subcore_barrier()

        def body(j, carry):
            pltpu.sync_copy(onesv, hist.at[idxv.at[j]], add=True)
            return carry

        lax.fori_loop(0, ROWS_PT, body, 0)
        plsc.subcore_barrier()
        for k in range(NZ):
            def do_chunk(k=k):
                off = sid * STRIPE + k * CHK
                pltpu.sync_copy(hist.at[pl.ds(off, CHK)], dumpv)
                pltpu.sync_copy(dumpv, deg.at[outidx, pl.ds(off, CHK)])
            if k < 5:
                do_chunk()
            else:
                pl.when(sid < NTILE - 1)(do_chunk)

    @pl.when(core == 0)
    def _():
        run(src2d, 0)

    @pl.when(core == 1)
    def _():
        run(dst2d, 1)


@functools.cache
def _deg_kernel():
    return pl.kernel(
        _sc_deg_body,
        out_type=jax.ShapeDtypeStruct((2, N, CW), jnp.float32),
        mesh=_mesh(),
        scratch_types=[
            pltpu.VMEM_SHARED((SPAD, CW), jnp.float32),
            pltpu.VMEM((ROWS_PT, BW), jnp.int32),
            pltpu.VMEM((BW, CW), jnp.float32),
            pltpu.VMEM((CHK, CW), jnp.float32),
            pltpu.VMEM((CHK, CW), jnp.float32),
        ],
    )


def _sc_agg_body(table, src2d, dst2d, z128, agg, acc, srcv, dstv, r0, r1,
                 sem0, sem1):
    core = lax.axis_index("c")
    sid = lax.axis_index("s")

    def one_pass(ch):
        tbl = table.at[ch]
        out = agg.at[ch]
        pltpu.sync_copy(z128, r1.at[pl.ds(0, CHK)])
        for k in range(NZ):
            pltpu.sync_copy(r1.at[pl.ds(0, CHK)],
                            acc.at[pl.ds(sid * STRIPE + k * CHK, CHK)])
        plsc.subcore_barrier()

        for half in range(2):
            base = sid * ROWS_PT + half * HROWS
            pltpu.sync_copy(src2d.at[pl.ds(base, HROWS)], srcv)
            pltpu.sync_copy(dst2d.at[pl.ds(base, HROWS)], dstv)
            pltpu.async_copy(tbl.at[srcv.at[0]], r0, sem0)

            def body(jj, carry):
                j0 = 2 * jj
                pltpu.make_async_copy(tbl.at[srcv.at[j0]], r0, sem0).wait()
                pltpu.async_copy(tbl.at[srcv.at[j0 + 1]], r1, sem1)
                pltpu.sync_copy(r0, acc.at[dstv.at[j0]], add=True)
                pltpu.make_async_copy(tbl.at[srcv.at[j0 + 1]], r1,
                                      sem1).wait()

                @pl.when(jj < HROWS // 2 - 1)
                def _():
                    pltpu.async_copy(tbl.at[srcv.at[j0 + 2]], r0, sem0)

                pltpu.sync_copy(r1, acc.at[dstv.at[j0 + 1]], add=True)
                return carry

            lax.fori_loop(0, HROWS // 2, body, 0)
        plsc.subcore_barrier()
        for k in range(NZ):
            def do_chunk(k=k):
                off = sid * STRIPE + k * CHK
                pltpu.sync_copy(acc.at[pl.ds(off, CHK)], r1.at[pl.ds(0, CHK)])
                pltpu.sync_copy(r1.at[pl.ds(0, CHK)], out.at[pl.ds(off, CHK)])
            if k < 5:
                do_chunk()
            else:
                pl.when(sid < NTILE - 1)(do_chunk)
        plsc.subcore_barrier()

    for cc in range(2):
        @pl.when(core == 0)
        def _(cc=cc):
            one_pass(cc)

        @pl.when(core == 1)
        def _(cc=cc):
            one_pass(2 + cc)


@functools.cache
def _agg_kernel():
    return pl.kernel(
        _sc_agg_body,
        out_type=jax.ShapeDtypeStruct((NCH, N, CW), jnp.float32),
        mesh=_mesh(),
        scratch_types=[
            pltpu.VMEM_SHARED((SPAD, CW), jnp.float32),
            pltpu.VMEM((HROWS, BW), jnp.int32),
            pltpu.VMEM((HROWS, BW), jnp.int32),
            pltpu.VMEM((BW, CW), jnp.float32),
            pltpu.VMEM((BW, CW), jnp.float32),
            pltpu.SemaphoreType.DMA,
            pltpu.SemaphoreType.DMA,
        ],
    )


def _norm_from(degblk):
    return lax.rsqrt(jnp.clip(degblk, 1.0, None))


def _tc_in_body(x, w, b, deg, h0, hs):
    acc = jnp.dot(x[...], w[...], preferred_element_type=jnp.float32) + b[...]
    h0[...] = acc
    onorm = _norm_from(deg[0, :, 0:1])
    s = acc * onorm
    for c in range(NCH):
        hs[c] = s[:, c * CW:(c + 1) * CW]


def _tc_conv_body(agg, w, g, bt, deg, h_old, hs, h_new, zbuf, stats, ab,
                  residual):
    p = pl.program_id(0)
    i = pl.program_id(1)

    @pl.when(p == 0)
    def _():
        @pl.when(i == 0)
        def _():
            stats[...] = jnp.zeros_like(stats)

        acc = jnp.dot(agg[0], w[0:CW, :], preferred_element_type=jnp.float32)
        for c in range(1, NCH):
            acc += jnp.dot(agg[c], w[c * CW:(c + 1) * CW, :],
                           preferred_element_type=jnp.float32)
        z = acc * _norm_from(deg[1, :, 0:1])
        zbuf[pl.ds(i * BN, BN), :] = z
        stats[0:1, :] += jnp.sum(z, axis=0, keepdims=True)
        stats[1:2, :] += jnp.sum(z * z, axis=0, keepdims=True)

    @pl.when(p == 1)
    def _():
        @pl.when(i == 0)
        def _():
            m = stats[0:1, :] * (1.0 / N)
            v = stats[1:2, :] * (1.0 / N) - m * m
            a = g[...] * lax.rsqrt(v + 1e-5)
            ab[0:1, :] = a
            ab[1:2, :] = bt[...] - m * a

        z = zbuf[pl.ds(i * BN, BN), :]
        x = jnp.maximum(z * ab[0:1, :] + ab[1:2, :], 0.0)
        if residual:
            x = h_old[...] + x
            h_new[...] = x
        s = x * _norm_from(deg[0, :, 0:1])
        for c in range(NCH):
            hs[c] = s[:, c * CW:(c + 1) * CW]


def _tc_head_body(h, wp, bp, wc, bc, pose, label, hsum):
    i = pl.program_id(0)

    @pl.when(i == 0)
    def _():
        hsum[...] = jnp.zeros_like(hsum)

    hblk = h[...]
    hsum[...] += jnp.sum(hblk, axis=0, keepdims=True)
    pp = jnp.dot(hblk, wp[...], preferred_element_type=jnp.float32) + bp[...]
    pose[...] = pp[:, :OUT]
    lab = jnp.dot(hsum[...] * (1.0 / N), wc[...],
                  preferred_element_type=jnp.float32) + bc[...]
    label[...] = lab


def _tc_input(x, w, b2, deg):
    return pl.pallas_call(
        _tc_in_body,
        grid=(NB,),
        in_specs=[
            pl.BlockSpec((BN, DIN), lambda i: (i, 0)),
            pl.BlockSpec((DIN, H), lambda i: (0, 0)),
            pl.BlockSpec((1, H), lambda i: (0, 0)),
            pl.BlockSpec((2, BN, CW), lambda i: (0, i, 0)),
        ],
        out_specs=[
            pl.BlockSpec((BN, H), lambda i: (i, 0)),
            pl.BlockSpec((NCH, BN, CW), lambda i: (0, i, 0)),
        ],
        out_shape=[
            jax.ShapeDtypeStruct((N, H), jnp.float32),
            jax.ShapeDtypeStruct((NCH, N, CW), jnp.float32),
        ],
    )(x, w, b2, deg)


def _tc_conv(agg, w, g2, bt2, deg, h_old, residual):
    body = functools.partial(_tc_conv_body, residual=residual)
    out_specs = [pl.BlockSpec((NCH, BN, CW), lambda p, i: (0, i, 0)),
                 pl.BlockSpec((BN, H), lambda p, i: (i, 0))]
    out_shape = [jax.ShapeDtypeStruct((NCH, N, CW), jnp.float32),
                 jax.ShapeDtypeStruct((N, H), jnp.float32)]
    if not residual:
        out_specs = out_specs[:1]
        out_shape = out_shape[:1]

        def body(agg, w, g, bt, deg, h_old, hs, zbuf, stats, ab):
            return _tc_conv_body(agg, w, g, bt, deg, h_old, hs, None, zbuf,
                                 stats, ab, residual=False)

    return pl.pallas_call(
        body,
        grid=(2, NB),
        in_specs=[
            pl.BlockSpec((NCH, BN, CW), lambda p, i: (0, i, 0)),
            pl.BlockSpec((H, H), lambda p, i: (0, 0)),
            pl.BlockSpec((1, H), lambda p, i: (0, 0)),
            pl.BlockSpec((1, H), lambda p, i: (0, 0)),
            pl.BlockSpec((2, BN, CW), lambda p, i: (0, i, 0)),
            pl.BlockSpec((BN, H), lambda p, i: (i, 0)),
        ],
        out_specs=out_specs,
        out_shape=out_shape,
        scratch_shapes=[
            pltpu.VMEM((N, H), jnp.float32),
            pltpu.VMEM((2, H), jnp.float32),
            pltpu.VMEM((2, H), jnp.float32),
        ],
    )(agg, w, g2, bt2, deg, h_old)


def _tc_head(h, wp, bp2, wc, bc2):
    return pl.pallas_call(
        _tc_head_body,
        grid=(NB,),
        in_specs=[
            pl.BlockSpec((BN, H), lambda i: (i, 0)),
            pl.BlockSpec((H, 128), lambda i: (0, 0)),
            pl.BlockSpec((1, 128), lambda i: (0, 0)),
            pl.BlockSpec((H, NC), lambda i: (0, 0)),
            pl.BlockSpec((1, NC), lambda i: (0, 0)),
        ],
        out_specs=[
            pl.BlockSpec((BN, OUT), lambda i: (i, 0)),
            pl.BlockSpec((1, NC), lambda i: (0, 0)),
        ],
        out_shape=[
            jax.ShapeDtypeStruct((N, OUT), jnp.float32),
            jax.ShapeDtypeStruct((1, NC), jnp.float32),
        ],
        scratch_shapes=[pltpu.VMEM((1, H), jnp.float32)],
    )(h, wp, bp2, wc, bc2)


def kernel(node_features, edge_index, W_in, b_in, W_conv, b_conv, gamma, beta,
           W_pose, b_pose, W_cls, b_cls):
    src2d = edge_index[0].reshape(EROWS, BW)
    dst2d = edge_index[1].reshape(EROWS, BW)
    ones_b = jnp.ones((BW, CW), jnp.float32)
    zdeg = jnp.zeros((CHK, CW), jnp.float32)
    z128 = jnp.zeros((CHK, CW), jnp.float32)

    deg = _deg_kernel()(src2d, dst2d, ones_b, zdeg)

    h, hs = _tc_input(node_features, W_in, b_in.reshape(1, H), deg)

    for blk in range(6):
        i0 = 2 * blk
        agg = _agg_kernel()(hs, src2d, dst2d, z128)
        xs = _tc_conv(agg, W_conv[i0], gamma[i0].reshape(1, H),
                      beta[i0].reshape(1, H), deg, h, residual=False)[0]
        agg2 = _agg_kernel()(xs, src2d, dst2d, z128)
        hs, h = _tc_conv(agg2, W_conv[i0 + 1], gamma[i0 + 1].reshape(1, H),
                         beta[i0 + 1].reshape(1, H), deg, h, residual=True)

    wp_pad = jnp.zeros((H, 128), jnp.float32).at[:, :OUT].set(W_pose)
    bp_pad = jnp.zeros((1, 128), jnp.float32).at[0, :OUT].set(b_pose)
    pose, label = _tc_head(h, wp_pad, bp_pad, W_cls, b_cls.reshape(1, NC))
    return (pose, label)
